# baseline (speedup 1.0000x reference)
import functools

import jax
import jax.numpy as jnp
from jax import lax
from jax.experimental import pallas as pl
from jax.experimental.pallas import tpu as pltpu
from jax.experimental.pallas import tpu_sc as plsc


def _encode_body(x_ref, feW1_ref, feb1_ref, feW2_ref, feb2_ref, projW_ref,
                 projb_ref, cb_ref, idx_ref, cba_ref, e2_ref, *,
                 n_cat, levels, feats, k_codes):
    f32 = jnp.float32

    @pl.when(pl.program_id(0) == 0)
    def _init():
        for c in range(n_cat):
            for l in range(levels):
                j = c * levels + l
                cb = cb_ref[c, l]
                cba_ref[j] = jnp.concatenate(
                    [jnp.ones_like(cb), cb], axis=1)
                e2_ref[j, :] = jnp.sum(cb * cb, axis=-1)

    x = x_ref[...]
    blk = x.shape[0]
    for c in range(n_cat):
        xc = x[:, c * feats:(c + 1) * feats]
        h = jnp.dot(xc, feW1_ref[c], preferred_element_type=f32)
        h = jnp.maximum(h + feb1_ref[c:c + 1, :], 0.0)
        emb = jnp.dot(h, feW2_ref[c], preferred_element_type=f32)
        emb = emb + feb2_ref[c:c + 1, :]
        for l in range(levels):
            z = jnp.dot(emb, projW_ref[c, l], preferred_element_type=f32)
            z = z + projb_ref[c, l:l + 1, :]
            j = c * levels + l
            za = jnp.concatenate([z * z, -2.0 * z], axis=1)
            dist = lax.dot_general(
                za, cba_ref[j], (((1,), (1,)), ((), ())),
                preferred_element_type=f32)
            dist = dist + e2_ref[j, :][None, :]
            idx = jnp.argmin(dist, axis=-1).astype(jnp.int32)
            rows = blk // 128
            idx_ref[j, pl.ds(pl.program_id(0) * rows, rows)] = (
                idx + j * k_codes).reshape(rows, 128)


def _encode(x, fe_W1, fe_b1, fe_W2, fe_b2, proj_W, proj_b, codebooks,
            chunk_b, start_blk, blk):
    bsz, in_dim = x.shape
    n_cat, feats, _ = fe_W1.shape
    levels, k_codes = codebooks.shape[1], codebooks.shape[2]
    n_lv = n_cat * levels
    steps = chunk_b // blk
    grid = (steps,)

    def rep(shape):
        return pl.BlockSpec(shape, lambda i: (0,) * len(shape))

    body = functools.partial(_encode_body, n_cat=n_cat, levels=levels,
                             feats=feats, k_codes=k_codes)
    return pl.pallas_call(
        body,
        grid=grid,
        in_specs=[
            pl.BlockSpec((blk, in_dim), lambda i, s0=start_blk: (i + s0, 0)),
            rep(fe_W1.shape), rep(fe_b1.shape),
            rep(fe_W2.shape), rep(fe_b2.shape),
            rep(proj_W.shape), rep(proj_b.shape),
            rep(codebooks.shape),
        ],
        out_specs=pl.BlockSpec((n_lv, chunk_b // 128, 128),
                               lambda i: (0, 0, 0)),
        out_shape=jax.ShapeDtypeStruct((n_lv, chunk_b // 128, 128),
                                       jnp.int32),
        scratch_shapes=[
            pltpu.VMEM((n_lv, k_codes, 2 * codebooks.shape[3]), jnp.float32),
            pltpu.VMEM((n_lv, k_codes), jnp.float32),
        ],
    )(x, fe_W1, fe_b1, fe_W2, fe_b2, proj_W, proj_b, codebooks)


def _sc_gather_scatter(table, idx2d, d, seg, chunk_b):
    n_rows = idx2d.shape[0]
    chunk = idx2d.shape[1]
    n = n_rows * chunk
    rows_per_j = chunk_b // chunk
    info = plsc.get_sparse_core_info()
    nc, ns = info.num_cores, info.num_subcores
    nw = nc * ns
    n_chunks = n_rows // nw
    n_lanes = info.num_lanes
    mesh = plsc.VectorSubcoreMesh(core_axis_name="c", subcore_axis_name="s")

    @functools.partial(
        pl.kernel, mesh=mesh,
        compiler_params=pltpu.CompilerParams(use_tc_tiling_on_sc=False),
        out_type=jax.ShapeDtypeStruct((n, d), jnp.float32),
        scratch_types=[
            pltpu.VMEM((n_chunks, chunk), jnp.int32),
            pltpu.VMEM((n_chunks, chunk), jnp.int32),
            pltpu.VMEM((n_chunks * chunk, d), jnp.float32),
            pltpu.SemaphoreType.DMA,
            pltpu.SemaphoreType.DMA,
        ],
    )
    def gather_k(table_hbm, idx_hbm, out_hbm, idx_v, pos_v, rows_v, gsem,
                 ssem):
        wid = lax.axis_index("s") * nc + lax.axis_index("c")
        base_row = wid * n_chunks
        pltpu.sync_copy(idx_hbm.at[pl.ds(base_row, n_chunks)], idx_v)
        half = n_chunks // 2

        def fire_gathers(lo, hi):
            return [
                pltpu.async_copy(table_hbm.at[idx_v.at[jj]],
                                 rows_v.at[pl.ds(jj * chunk, chunk)], gsem)
                for jj in range(lo, hi)
            ]

        def fire_scatters(lo, hi):
            return [
                pltpu.async_copy(rows_v.at[pl.ds(jj * chunk, chunk)],
                                 out_hbm.at[pos_v.at[jj]], ssem)
                for jj in range(lo, hi)
            ]

        g0 = fire_gathers(0, half)
        g1 = fire_gathers(half, n_chunks)
        for jj in range(n_chunks):
            r = base_row + jj
            j = r // rows_per_j
            m = r - j * rows_per_j
            base = ((j // seg) * (seg * chunk_b) + m * (chunk * seg)
                    + (j % seg))
            for v in range(chunk // n_lanes):
                lane = lax.iota(jnp.int32, n_lanes)
                pos_v[jj, v * n_lanes:(v + 1) * n_lanes] = (
                    base + seg * (v * n_lanes + lane))
        for cp in g0:
            cp.wait()
        s0 = fire_scatters(0, half)
        for cp in g1:
            cp.wait()
        s1 = fire_scatters(half, n_chunks)
        for cp in s0 + s1:
            cp.wait()

    return gather_k(table, idx2d)


def _decode_body(q_ref, decW1_ref, decb1_ref, decW2_ref, decb2_ref, out_ref):
    f32 = jnp.float32
    n_seg = q_ref.shape[0]
    h2 = decb1_ref[...]
    for s in range(n_seg):
        h2 = h2 + jnp.dot(q_ref[s], decW1_ref[s * 128:(s + 1) * 128, :],
                          preferred_element_type=f32)
    h2 = jnp.maximum(h2, 0.0)
    out = jnp.dot(h2, decW2_ref[...], preferred_element_type=f32)
    out_ref[...] = out + decb2_ref[...]


def _decode(q6, dec_W1, dec_b1, dec_W2, dec_b2, blk):
    n_seg, bsz, seg_w = q6.shape
    out_d = dec_W2.shape[1]
    grid = (bsz // blk,)

    def rep(shape):
        return pl.BlockSpec(shape, lambda i: (0,) * len(shape))

    return pl.pallas_call(
        _decode_body,
        grid=grid,
        in_specs=[
            pl.BlockSpec((n_seg, blk, seg_w), lambda i: (0, i, 0)),
            rep(dec_W1.shape), rep((1, dec_b1.shape[0])),
            rep(dec_W2.shape), rep((1, dec_b2.shape[0])),
        ],
        out_specs=pl.BlockSpec((blk, out_d), lambda i: (i, 0)),
        out_shape=jax.ShapeDtypeStruct((bsz, out_d), jnp.float32),
    )(q6, dec_W1, dec_b1.reshape(1, -1), dec_W2, dec_b2.reshape(1, -1))


def kernel(x, fe_W1, fe_b1, fe_W2, fe_b2, proj_W, proj_b, codebooks,
           dec_W1, dec_b1, dec_W2, dec_b2):
    bsz = x.shape[0]
    n_cat, levels, k_codes, d = codebooks.shape
    n_lv = n_cat * levels
    seg = 128 // d
    table = codebooks.reshape(n_lv * k_codes, d)
    blk = 512
    chunks = [(0, 3 * bsz // 4), (3 * bsz // 4, bsz // 4)]

    outs = []
    for start, chunk_b in chunks:
        idx = _encode(x, fe_W1, fe_b1, fe_W2, fe_b2, proj_W, proj_b,
                      codebooks, chunk_b, start // blk, blk=blk)
        idx2d = idx.reshape(n_lv * chunk_b // 128, 128)
        q = _sc_gather_scatter(table, idx2d, d, seg, chunk_b)
        q6 = q.reshape(n_lv * d // 128, chunk_b, 128)
        outs.append(_decode(q6, dec_W1, dec_b1, dec_W2, dec_b2, blk=blk))
    return jnp.concatenate(outs, axis=0)


# speedup vs baseline: 1.0655x; 1.0655x over previous
import functools

import jax
import jax.numpy as jnp
from jax import lax
from jax.experimental import pallas as pl
from jax.experimental.pallas import tpu as pltpu
from jax.experimental.pallas import tpu_sc as plsc


def _encode_body(x_ref, feW1_ref, feb1_ref, feW2_ref, feb2_ref, projW_ref,
                 projb_ref, cb_ref, idx_ref, cba_ref, e2_ref, *,
                 n_cat, levels, feats, k_codes):
    f32 = jnp.float32

    @pl.when(pl.program_id(0) == 0)
    def _init():
        for c in range(n_cat):
            for l in range(levels):
                j = c * levels + l
                cb = cb_ref[c, l]
                cba_ref[j] = jnp.concatenate(
                    [jnp.ones_like(cb), cb], axis=1)
                e2_ref[j, :] = jnp.sum(cb * cb, axis=-1)

    x = x_ref[...]
    blk = x.shape[0]
    for c in range(n_cat):
        xc = x[:, c * feats:(c + 1) * feats]
        h = jnp.dot(xc, feW1_ref[c], preferred_element_type=f32)
        h = jnp.maximum(h + feb1_ref[c:c + 1, :], 0.0)
        emb = jnp.dot(h, feW2_ref[c], preferred_element_type=f32)
        emb = emb + feb2_ref[c:c + 1, :]
        for l in range(levels):
            z = jnp.dot(emb, projW_ref[c, l], preferred_element_type=f32)
            z = z + projb_ref[c, l:l + 1, :]
            j = c * levels + l
            za = jnp.concatenate([z * z, -2.0 * z], axis=1)
            dist = lax.dot_general(
                za, cba_ref[j], (((1,), (1,)), ((), ())),
                preferred_element_type=f32)
            dist = dist + e2_ref[j, :][None, :]
            idx = jnp.argmin(dist, axis=-1).astype(jnp.int32)
            rows = blk // 128
            idx_ref[j, pl.ds(pl.program_id(0) * rows, rows)] = (
                idx + j * k_codes).reshape(rows, 128)


def _encode(x, fe_W1, fe_b1, fe_W2, fe_b2, proj_W, proj_b, codebooks,
            chunk_b, start_blk, blk):
    bsz, in_dim = x.shape
    n_cat, feats, _ = fe_W1.shape
    levels, k_codes = codebooks.shape[1], codebooks.shape[2]
    n_lv = n_cat * levels
    steps = chunk_b // blk
    grid = (steps,)

    def rep(shape):
        return pl.BlockSpec(shape, lambda i: (0,) * len(shape))

    body = functools.partial(_encode_body, n_cat=n_cat, levels=levels,
                             feats=feats, k_codes=k_codes)
    return pl.pallas_call(
        body,
        grid=grid,
        in_specs=[
            pl.BlockSpec((blk, in_dim), lambda i, s0=start_blk: (i + s0, 0)),
            rep(fe_W1.shape), rep(fe_b1.shape),
            rep(fe_W2.shape), rep(fe_b2.shape),
            rep(proj_W.shape), rep(proj_b.shape),
            rep(codebooks.shape),
        ],
        out_specs=pl.BlockSpec((n_lv, chunk_b // 128, 128),
                               lambda i: (0, 0, 0)),
        out_shape=jax.ShapeDtypeStruct((n_lv, chunk_b // 128, 128),
                                       jnp.int32),
        scratch_shapes=[
            pltpu.VMEM((n_lv, k_codes, 2 * codebooks.shape[3]), jnp.float32),
            pltpu.VMEM((n_lv, k_codes), jnp.float32),
        ],
    )(x, fe_W1, fe_b1, fe_W2, fe_b2, proj_W, proj_b, codebooks)


def _sc_gather_scatter(table, idx2d, d, seg, chunk_b):
    n_rows = idx2d.shape[0]
    chunk = idx2d.shape[1]
    n = n_rows * chunk
    rows_per_j = chunk_b // chunk
    info = plsc.get_sparse_core_info()
    nc, ns = info.num_cores, info.num_subcores
    nw = nc * ns
    n_chunks = n_rows // nw
    n_lanes = info.num_lanes
    mesh = plsc.VectorSubcoreMesh(core_axis_name="c", subcore_axis_name="s")

    @functools.partial(
        pl.kernel, mesh=mesh,
        compiler_params=pltpu.CompilerParams(use_tc_tiling_on_sc=False),
        out_type=jax.ShapeDtypeStruct((n, d), jnp.float32),
        scratch_types=[
            pltpu.VMEM((n_chunks, chunk), jnp.int32),
            pltpu.VMEM((n_chunks, chunk), jnp.int32),
            pltpu.VMEM((n_chunks * chunk, d), jnp.float32),
            pltpu.SemaphoreType.DMA,
            pltpu.SemaphoreType.DMA,
        ],
    )
    def gather_k(table_hbm, idx_hbm, out_hbm, idx_v, pos_v, rows_v, gsem,
                 ssem):
        wid = lax.axis_index("s") * nc + lax.axis_index("c")
        base_row = wid * n_chunks
        pltpu.sync_copy(idx_hbm.at[pl.ds(base_row, n_chunks)], idx_v)
        half = n_chunks // 2

        def fire_gathers(lo, hi):
            return [
                pltpu.async_copy(table_hbm.at[idx_v.at[jj]],
                                 rows_v.at[pl.ds(jj * chunk, chunk)], gsem)
                for jj in range(lo, hi)
            ]

        def fire_scatters(lo, hi):
            return [
                pltpu.async_copy(rows_v.at[pl.ds(jj * chunk, chunk)],
                                 out_hbm.at[pos_v.at[jj]], ssem)
                for jj in range(lo, hi)
            ]

        g0 = fire_gathers(0, half)
        g1 = fire_gathers(half, n_chunks)
        for jj in range(n_chunks):
            r = base_row + jj
            j = r // rows_per_j
            m = r - j * rows_per_j
            base = ((j // seg) * (seg * chunk_b) + m * (chunk * seg)
                    + (j % seg))
            for v in range(chunk // n_lanes):
                lane = lax.iota(jnp.int32, n_lanes)
                pos_v[jj, v * n_lanes:(v + 1) * n_lanes] = (
                    base + seg * (v * n_lanes + lane))
        for cp in g0:
            cp.wait()
        s0 = fire_scatters(0, half)
        for cp in g1:
            cp.wait()
        s1 = fire_scatters(half, n_chunks)
        for cp in s0 + s1:
            cp.wait()

    return gather_k(table, idx2d)


def _decode_body(q_ref, decW1_ref, decb1_ref, decW2_ref, decb2_ref, out_ref):
    f32 = jnp.float32
    n_seg = q_ref.shape[0]
    h2 = decb1_ref[...]
    for s in range(n_seg):
        h2 = h2 + jnp.dot(q_ref[s], decW1_ref[s * 128:(s + 1) * 128, :],
                          preferred_element_type=f32)
    h2 = jnp.maximum(h2, 0.0)
    out = jnp.dot(h2, decW2_ref[...], preferred_element_type=f32)
    out_ref[...] = out + decb2_ref[...]


def _decode(q6, dec_W1, dec_b1, dec_W2, dec_b2, blk):
    n_seg, bsz, seg_w = q6.shape
    out_d = dec_W2.shape[1]
    grid = (bsz // blk,)

    def rep(shape):
        return pl.BlockSpec(shape, lambda i: (0,) * len(shape))

    return pl.pallas_call(
        _decode_body,
        grid=grid,
        in_specs=[
            pl.BlockSpec((n_seg, blk, seg_w), lambda i: (0, i, 0)),
            rep(dec_W1.shape), rep((1, dec_b1.shape[0])),
            rep(dec_W2.shape), rep((1, dec_b2.shape[0])),
        ],
        out_specs=pl.BlockSpec((blk, out_d), lambda i: (i, 0)),
        out_shape=jax.ShapeDtypeStruct((bsz, out_d), jnp.float32),
    )(q6, dec_W1, dec_b1.reshape(1, -1), dec_W2, dec_b2.reshape(1, -1))


def kernel(x, fe_W1, fe_b1, fe_W2, fe_b2, proj_W, proj_b, codebooks,
           dec_W1, dec_b1, dec_W2, dec_b2):
    bsz = x.shape[0]
    n_cat, levels, k_codes, d = codebooks.shape
    n_lv = n_cat * levels
    seg = 128 // d
    table = codebooks.reshape(n_lv * k_codes, d)
    blk = 512
    chunks = [(0, bsz // 2), (bsz // 2, bsz // 2)]

    outs = []
    for start, chunk_b in chunks:
        idx = _encode(x, fe_W1, fe_b1, fe_W2, fe_b2, proj_W, proj_b,
                      codebooks, chunk_b, start // blk, blk=blk)
        idx2d = idx.reshape(n_lv * chunk_b // 128, 128)
        q = _sc_gather_scatter(table, idx2d, d, seg, chunk_b)
        q6 = q.reshape(n_lv * d // 128, chunk_b, 128)
        outs.append(_decode(q6, dec_W1, dec_b1, dec_W2, dec_b2, blk=blk))
    return jnp.concatenate(outs, axis=0)


# speedup vs baseline: 1.3223x; 1.2410x over previous
import functools

import jax
import jax.numpy as jnp
from jax import lax
from jax.experimental import pallas as pl
from jax.experimental.pallas import tpu as pltpu
from jax.experimental.pallas import tpu_sc as plsc


def _encode_body(x_ref, feW1_ref, feb1_ref, feW2T_ref, feb2_ref, projWT_ref,
                 projb_ref, cbT_ref, idx_ref, cba_ref, e2_ref, *,
                 n_cat, levels, feats, k_codes):
    f32 = jnp.float32

    @pl.when(pl.program_id(0) == 0)
    def _init():
        for c in range(n_cat):
            for l in range(levels):
                j = c * levels + l
                cbT = cbT_ref[c, l]
                cba_ref[j] = jnp.concatenate(
                    [jnp.ones_like(cbT), cbT], axis=0)
                e2_ref[j, :] = jnp.sum(cbT * cbT, axis=0)

    x = x_ref[...]
    blk = x.shape[0]
    for c in range(n_cat):
        xc = x[:, c * feats:(c + 1) * feats]
        h = jnp.dot(xc, feW1_ref[c], preferred_element_type=f32)
        h = jnp.maximum(h + feb1_ref[c:c + 1, :], 0.0)
        emb = lax.dot_general(
            h, feW2T_ref[c], (((1,), (1,)), ((), ())),
            preferred_element_type=f32)
        emb = emb + feb2_ref[c:c + 1, :]
        for l in range(levels):
            z = lax.dot_general(
                emb, projWT_ref[c, l], (((1,), (1,)), ((), ())),
                preferred_element_type=f32)
            z = z + projb_ref[c, l:l + 1, :]
            j = c * levels + l
            za = jnp.concatenate([z * z, -2.0 * z], axis=1)
            dist = lax.dot_general(
                za, cba_ref[j], (((1,), (0,)), ((), ())),
                preferred_element_type=f32)
            dist = dist + e2_ref[j, :][None, :]
            idx = jnp.argmin(dist, axis=-1).astype(jnp.int32)
            rows = blk // 128
            idx_ref[j, pl.ds(pl.program_id(0) * rows, rows)] = (
                idx + j * k_codes).reshape(rows, 128)


def _encode(x, fe_W1, fe_b1, fe_W2T, fe_b2, proj_WT, proj_b, cbT,
            chunk_b, start_blk, blk):
    bsz, in_dim = x.shape
    n_cat, feats, _ = fe_W1.shape
    levels, d, k_codes = cbT.shape[1], cbT.shape[2], cbT.shape[3]
    n_lv = n_cat * levels
    steps = chunk_b // blk
    grid = (steps,)

    def rep(shape):
        return pl.BlockSpec(shape, lambda i: (0,) * len(shape))

    body = functools.partial(_encode_body, n_cat=n_cat, levels=levels,
                             feats=feats, k_codes=k_codes)
    return pl.pallas_call(
        body,
        grid=grid,
        in_specs=[
            pl.BlockSpec((blk, in_dim), lambda i, s0=start_blk: (i + s0, 0)),
            rep(fe_W1.shape), rep(fe_b1.shape),
            rep(fe_W2T.shape), rep(fe_b2.shape),
            rep(proj_WT.shape), rep(proj_b.shape),
            rep(cbT.shape),
        ],
        out_specs=pl.BlockSpec((n_lv, chunk_b // 128, 128),
                               lambda i: (0, 0, 0)),
        out_shape=jax.ShapeDtypeStruct((n_lv, chunk_b // 128, 128),
                                       jnp.int32),
        scratch_shapes=[
            pltpu.VMEM((n_lv, 2 * d, k_codes), jnp.float32),
            pltpu.VMEM((n_lv, k_codes), jnp.float32),
        ],
    )(x, fe_W1, fe_b1, fe_W2T, fe_b2, proj_WT, proj_b, cbT)


def _sc_gather_scatter(table, idx2d, d, seg, chunk_b):
    n_rows = idx2d.shape[0]
    chunk = idx2d.shape[1]
    n = n_rows * chunk
    rows_per_j = chunk_b // chunk
    info = plsc.get_sparse_core_info()
    nc, ns = info.num_cores, info.num_subcores
    nw = nc * ns
    n_chunks = n_rows // nw
    n_lanes = info.num_lanes
    mesh = plsc.VectorSubcoreMesh(core_axis_name="c", subcore_axis_name="s")

    @functools.partial(
        pl.kernel, mesh=mesh,
        compiler_params=pltpu.CompilerParams(use_tc_tiling_on_sc=False),
        out_type=jax.ShapeDtypeStruct((n, d), jnp.float32),
        scratch_types=[
            pltpu.VMEM((n_chunks, chunk), jnp.int32),
            pltpu.VMEM((n_chunks, chunk), jnp.int32),
            pltpu.VMEM((n_chunks * chunk, d), jnp.float32),
            pltpu.SemaphoreType.DMA,
            pltpu.SemaphoreType.DMA,
        ],
    )
    def gather_k(table_hbm, idx_hbm, out_hbm, idx_v, pos_v, rows_v, gsem,
                 ssem):
        wid = lax.axis_index("s") * nc + lax.axis_index("c")
        base_row = wid * n_chunks
        pltpu.sync_copy(idx_hbm.at[pl.ds(base_row, n_chunks)], idx_v)
        half = n_chunks // 2

        def fire_gathers(lo, hi):
            return [
                pltpu.async_copy(table_hbm.at[idx_v.at[jj]],
                                 rows_v.at[pl.ds(jj * chunk, chunk)], gsem)
                for jj in range(lo, hi)
            ]

        def fire_scatters(lo, hi):
            return [
                pltpu.async_copy(rows_v.at[pl.ds(jj * chunk, chunk)],
                                 out_hbm.at[pos_v.at[jj]], ssem)
                for jj in range(lo, hi)
            ]

        g0 = fire_gathers(0, half)
        g1 = fire_gathers(half, n_chunks)
        for jj in range(n_chunks):
            r = base_row + jj
            j = r // rows_per_j
            m = r - j * rows_per_j
            base = ((j // seg) * (seg * chunk_b) + m * (chunk * seg)
                    + (j % seg))
            for v in range(chunk // n_lanes):
                lane = lax.iota(jnp.int32, n_lanes)
                pos_v[jj, v * n_lanes:(v + 1) * n_lanes] = (
                    base + seg * (v * n_lanes + lane))
        for cp in g0:
            cp.wait()
        s0 = fire_scatters(0, half)
        for cp in g1:
            cp.wait()
        s1 = fire_scatters(half, n_chunks)
        for cp in s0 + s1:
            cp.wait()

    return gather_k(table, idx2d)


def _decode_body(q_ref, decW1_ref, decb1_ref, decW2_ref, decb2_ref, out_ref):
    f32 = jnp.float32
    n_seg = q_ref.shape[0]
    h2 = decb1_ref[...]
    for s in range(n_seg):
        h2 = h2 + jnp.dot(q_ref[s], decW1_ref[s * 128:(s + 1) * 128, :],
                          preferred_element_type=f32)
    h2 = jnp.maximum(h2, 0.0)
    out = jnp.dot(h2, decW2_ref[...], preferred_element_type=f32)
    out_ref[...] = out + decb2_ref[...]


def _decode(q6, dec_W1, dec_b1, dec_W2, dec_b2, blk):
    n_seg, bsz, seg_w = q6.shape
    out_d = dec_W2.shape[1]
    grid = (bsz // blk,)

    def rep(shape):
        return pl.BlockSpec(shape, lambda i: (0,) * len(shape))

    return pl.pallas_call(
        _decode_body,
        grid=grid,
        in_specs=[
            pl.BlockSpec((n_seg, blk, seg_w), lambda i: (0, i, 0)),
            rep(dec_W1.shape), rep((1, dec_b1.shape[0])),
            rep(dec_W2.shape), rep((1, dec_b2.shape[0])),
        ],
        out_specs=pl.BlockSpec((blk, out_d), lambda i: (i, 0)),
        out_shape=jax.ShapeDtypeStruct((bsz, out_d), jnp.float32),
    )(q6, dec_W1, dec_b1.reshape(1, -1), dec_W2, dec_b2.reshape(1, -1))


def kernel(x, fe_W1, fe_b1, fe_W2, fe_b2, proj_W, proj_b, codebooks,
           dec_W1, dec_b1, dec_W2, dec_b2):
    bsz = x.shape[0]
    n_cat, levels, k_codes, d = codebooks.shape
    n_lv = n_cat * levels
    seg = 128 // d
    table = codebooks.reshape(n_lv * k_codes, d)
    fe_W2T = jnp.swapaxes(fe_W2, 1, 2)
    proj_WT = jnp.swapaxes(proj_W, 2, 3)
    cbT = jnp.swapaxes(codebooks, 2, 3)
    blk = 512
    chunks = [(0, bsz // 2), (bsz // 2, bsz // 2)]

    outs = []
    for start, chunk_b in chunks:
        idx = _encode(x, fe_W1, fe_b1, fe_W2T, fe_b2, proj_WT, proj_b,
                      cbT, chunk_b, start // blk, blk=blk)
        idx2d = idx.reshape(n_lv * chunk_b // 128, 128)
        q = _sc_gather_scatter(table, idx2d, d, seg, chunk_b)
        q6 = q.reshape(n_lv * d // 128, chunk_b, 128)
        outs.append(_decode(q6, dec_W1, dec_b1, dec_W2, dec_b2, blk=blk))
    return jnp.concatenate(outs, axis=0)


# speedup vs baseline: 1.4467x; 1.0941x over previous
import functools

import jax
import jax.numpy as jnp
from jax import lax
from jax.experimental import pallas as pl
from jax.experimental.pallas import tpu as pltpu
from jax.experimental.pallas import tpu_sc as plsc


def _encode_body(x_ref, feW1_ref, feb1_ref, feW2T_ref, feb2_ref, projWT_ref,
                 projb_ref, cbT_ref, idx_ref, cba_ref, e2_ref, *,
                 n_cat, levels, feats, k_codes):
    f32 = jnp.float32

    @pl.when(pl.program_id(0) == 0)
    def _init():
        for c in range(n_cat):
            for l in range(levels):
                j = c * levels + l
                cbT = cbT_ref[c, l]
                cba_ref[j] = jnp.concatenate(
                    [jnp.ones_like(cbT), cbT], axis=0)
                e2_ref[j, :] = jnp.sum(cbT * cbT, axis=0)

    x = x_ref[...]
    blk = x.shape[0]
    for c in range(n_cat):
        xc = x[:, c * feats:(c + 1) * feats]
        h = jnp.dot(xc, feW1_ref[c], preferred_element_type=f32)
        h = jnp.maximum(h + feb1_ref[c:c + 1, :], 0.0)
        emb = lax.dot_general(
            h, feW2T_ref[c], (((1,), (1,)), ((), ())),
            preferred_element_type=f32)
        emb = emb + feb2_ref[c:c + 1, :]
        for l in range(levels):
            z = lax.dot_general(
                emb, projWT_ref[c, l], (((1,), (1,)), ((), ())),
                preferred_element_type=f32)
            z = z + projb_ref[c, l:l + 1, :]
            j = c * levels + l
            za = jnp.concatenate([z * z, -2.0 * z], axis=1)
            dist = lax.dot_general(
                za, cba_ref[j], (((1,), (0,)), ((), ())),
                preferred_element_type=f32)
            dist = dist + e2_ref[j, :][None, :]
            idx = jnp.argmin(dist, axis=-1).astype(jnp.int32)
            rows = blk // 128
            idx_ref[j, pl.ds(pl.program_id(0) * rows, rows)] = (
                idx + j * k_codes).reshape(rows, 128)


def _encode(x, fe_W1, fe_b1, fe_W2T, fe_b2, proj_WT, proj_b, cbT,
            chunk_b, start_blk, blk):
    bsz, in_dim = x.shape
    n_cat, feats, _ = fe_W1.shape
    levels, d, k_codes = cbT.shape[1], cbT.shape[2], cbT.shape[3]
    n_lv = n_cat * levels
    steps = chunk_b // blk
    grid = (steps,)

    def rep(shape):
        return pl.BlockSpec(shape, lambda i: (0,) * len(shape))

    body = functools.partial(_encode_body, n_cat=n_cat, levels=levels,
                             feats=feats, k_codes=k_codes)
    return pl.pallas_call(
        body,
        grid=grid,
        in_specs=[
            pl.BlockSpec((blk, in_dim), lambda i, s0=start_blk: (i + s0, 0)),
            rep(fe_W1.shape), rep(fe_b1.shape),
            rep(fe_W2T.shape), rep(fe_b2.shape),
            rep(proj_WT.shape), rep(proj_b.shape),
            rep(cbT.shape),
        ],
        out_specs=pl.BlockSpec((n_lv, chunk_b // 128, 128),
                               lambda i: (0, 0, 0)),
        out_shape=jax.ShapeDtypeStruct((n_lv, chunk_b // 128, 128),
                                       jnp.int32),
        scratch_shapes=[
            pltpu.VMEM((n_lv, 2 * d, k_codes), jnp.float32),
            pltpu.VMEM((n_lv, k_codes), jnp.float32),
        ],
    )(x, fe_W1, fe_b1, fe_W2T, fe_b2, proj_WT, proj_b, cbT)


def _sc_gather_scatter(table, idx2d, d, seg, chunk_b):
    n_rows = idx2d.shape[0]
    chunk = idx2d.shape[1]
    n = n_rows * chunk
    rows_per_j = chunk_b // chunk
    info = plsc.get_sparse_core_info()
    nc, ns = info.num_cores, info.num_subcores
    nw = nc * ns
    n_chunks = n_rows // nw
    n_lanes = info.num_lanes
    mesh = plsc.VectorSubcoreMesh(core_axis_name="c", subcore_axis_name="s")

    @functools.partial(
        pl.kernel, mesh=mesh,
        compiler_params=pltpu.CompilerParams(use_tc_tiling_on_sc=False),
        out_type=jax.ShapeDtypeStruct((n, d), jnp.float32),
        scratch_types=[
            pltpu.VMEM((n_chunks, chunk), jnp.int32),
            pltpu.VMEM((n_chunks, chunk), jnp.int32),
            pltpu.VMEM((n_chunks * chunk, d), jnp.float32),
            pltpu.SemaphoreType.DMA,
            pltpu.SemaphoreType.DMA,
        ],
    )
    def gather_k(table_hbm, idx_hbm, out_hbm, idx_v, pos_v, rows_v, gsem,
                 ssem):
        wid = lax.axis_index("s") * nc + lax.axis_index("c")
        base_row = wid * n_chunks
        pltpu.sync_copy(idx_hbm.at[pl.ds(base_row, n_chunks)], idx_v)
        half = n_chunks // 2

        def fire_gathers(lo, hi):
            return [
                pltpu.async_copy(table_hbm.at[idx_v.at[jj]],
                                 rows_v.at[pl.ds(jj * chunk, chunk)], gsem)
                for jj in range(lo, hi)
            ]

        def fire_scatters(lo, hi):
            return [
                pltpu.async_copy(rows_v.at[pl.ds(jj * chunk, chunk)],
                                 out_hbm.at[pos_v.at[jj]], ssem)
                for jj in range(lo, hi)
            ]

        g0 = fire_gathers(0, half)
        g1 = fire_gathers(half, n_chunks)
        for jj in range(n_chunks):
            r = base_row + jj
            j = r // rows_per_j
            m = r - j * rows_per_j
            base = ((j // seg) * (seg * chunk_b) + m * (chunk * seg)
                    + (j % seg))
            for v in range(chunk // n_lanes):
                lane = lax.iota(jnp.int32, n_lanes)
                pos_v[jj, v * n_lanes:(v + 1) * n_lanes] = (
                    base + seg * (v * n_lanes + lane))
        for cp in g0:
            cp.wait()
        s0 = fire_scatters(0, half)
        for cp in g1:
            cp.wait()
        s1 = fire_scatters(half, n_chunks)
        for cp in s0 + s1:
            cp.wait()

    return gather_k(table, idx2d)


def _decode_body(q_ref, decW1_ref, decb1_ref, decW2_ref, decb2_ref, out_ref):
    f32 = jnp.float32
    n_seg = q_ref.shape[0]
    h2 = decb1_ref[...]
    for s in range(n_seg):
        h2 = h2 + jnp.dot(q_ref[s], decW1_ref[s * 128:(s + 1) * 128, :],
                          preferred_element_type=f32)
    h2 = jnp.maximum(h2, 0.0)
    out = jnp.dot(h2, decW2_ref[...], preferred_element_type=f32)
    out_ref[...] = out + decb2_ref[...]


def _decode(q6, dec_W1, dec_b1, dec_W2, dec_b2, blk):
    n_seg, bsz, seg_w = q6.shape
    out_d = dec_W2.shape[1]
    grid = (bsz // blk,)

    def rep(shape):
        return pl.BlockSpec(shape, lambda i: (0,) * len(shape))

    return pl.pallas_call(
        _decode_body,
        grid=grid,
        in_specs=[
            pl.BlockSpec((n_seg, blk, seg_w), lambda i: (0, i, 0)),
            rep(dec_W1.shape), rep((1, dec_b1.shape[0])),
            rep(dec_W2.shape), rep((1, dec_b2.shape[0])),
        ],
        out_specs=pl.BlockSpec((blk, out_d), lambda i: (i, 0)),
        out_shape=jax.ShapeDtypeStruct((bsz, out_d), jnp.float32),
    )(q6, dec_W1, dec_b1.reshape(1, -1), dec_W2, dec_b2.reshape(1, -1))


def kernel(x, fe_W1, fe_b1, fe_W2, fe_b2, proj_W, proj_b, codebooks,
           dec_W1, dec_b1, dec_W2, dec_b2):
    bsz = x.shape[0]
    n_cat, levels, k_codes, d = codebooks.shape
    n_lv = n_cat * levels
    seg = 128 // d
    table = codebooks.reshape(n_lv * k_codes, d)
    fe_W2T = jnp.swapaxes(fe_W2, 1, 2)
    proj_WT = jnp.swapaxes(proj_W, 2, 3)
    cbT = jnp.swapaxes(codebooks, 2, 3)
    blk = 1024
    chunks = [(0, bsz // 2), (bsz // 2, bsz // 2)]

    outs = []
    for start, chunk_b in chunks:
        idx = _encode(x, fe_W1, fe_b1, fe_W2T, fe_b2, proj_WT, proj_b,
                      cbT, chunk_b, start // blk, blk=blk)
        idx2d = idx.reshape(n_lv * chunk_b // 128, 128)
        q = _sc_gather_scatter(table, idx2d, d, seg, chunk_b)
        q6 = q.reshape(n_lv * d // 128, chunk_b, 128)
        outs.append(_decode(q6, dec_W1, dec_b1, dec_W2, dec_b2, blk=blk))
    return jnp.concatenate(outs, axis=0)
